# baseline (speedup 1.0000x reference)
import jax
import jax.numpy as jnp
from jax import lax
from jax.experimental import pallas as pl
from jax.experimental.pallas import tpu as pltpu
from jax.experimental.pallas import tpu_sc as plsc

_N_CNSTRS = 3
_IDEAL = 4.0
_L = 16


def _sc_body(coords_hbm, off_hbm, out_hbm, coords_v, off_v, out_v):
    cid = lax.axis_index("c")
    sid = lax.axis_index("s")

    @pl.when(jnp.logical_and(cid == 0, sid == 0))
    def _():
        pltpu.sync_copy(coords_hbm, coords_v)
        pltpu.sync_copy(off_hbm, off_v)

        iota = lax.broadcasted_iota(jnp.int32, (_L,), 0)
        off1 = off_v[pl.ds(0, _L)]
        off2 = plsc.load_gather(off_v, [iota + 1])
        a1 = off1 + iota
        a2 = off2 + iota + 1

        b1 = a1 * 3
        b2 = a2 * 3
        dx = plsc.load_gather(coords_v, [b1]) - plsc.load_gather(coords_v, [b2])
        dy = plsc.load_gather(coords_v, [b1 + 1]) - plsc.load_gather(coords_v, [b2 + 1])
        dz = plsc.load_gather(coords_v, [b1 + 2]) - plsc.load_gather(coords_v, [b2 + 2])
        d2 = dx * dx + dy * dy + dz * dz

        bits = plsc.bitcast(d2, jnp.int32)
        y = plsc.bitcast(jnp.int32(0x5F3759DF) - (bits >> 1), jnp.float32)
        half_d2 = 0.5 * d2
        for _unused in range(3):
            y = y * (1.5 - half_d2 * y * y)
        dist = d2 * y

        h = dist - _IDEAL
        h = h * h
        h = jnp.where(iota < _N_CNSTRS, h, 0.0)
        total = jnp.sum(h)

        out_v[...] = lax.broadcast(total, (_L,))
        pltpu.sync_copy(out_v, out_hbm)


def kernel(coords, pose_stack_block_coord_offset):
    n_atoms = coords.shape[1]
    n_blocks = pose_stack_block_coord_offset.shape[1]
    c = coords.reshape(n_atoms * 3)
    off = pose_stack_block_coord_offset.reshape(n_blocks)

    mesh = plsc.VectorSubcoreMesh(core_axis_name="c", subcore_axis_name="s")
    res = pl.kernel(
        _sc_body,
        out_type=jax.ShapeDtypeStruct((_L,), jnp.float32),
        mesh=mesh,
        compiler_params=pltpu.CompilerParams(needs_layout_passes=False),
        scratch_types=[
            pltpu.VMEM((n_atoms * 3,), jnp.float32),
            pltpu.VMEM((n_blocks,), jnp.int32),
            pltpu.VMEM((_L,), jnp.float32),
        ],
    )(c, off)
    return res[:1].reshape(1, 1)


# speedup vs baseline: 1.1116x; 1.1116x over previous
import jax
import jax.numpy as jnp
from jax import lax
from jax.experimental import pallas as pl
from jax.experimental.pallas import tpu as pltpu
from jax.experimental.pallas import tpu_sc as plsc

_N_CNSTRS = 3
_IDEAL = 4.0
_L = 16


def _sc_body(coords_hbm, off_hbm, out_hbm, off_v, idx_v, gat_v, out_v, sem):
    pltpu.sync_copy(off_hbm.at[pl.ds(0, 32)], off_v)

    iota = lax.broadcasted_iota(jnp.int32, (_L,), 0)
    off1 = off_v[pl.ds(0, _L)]
    off2 = off_v[pl.ds(1, _L)]
    b1 = (off1 + iota) * 3
    b2 = (off2 + iota + 1) * 3
    idx_v[pl.ds(0, _L)] = b1
    idx_v[pl.ds(_L, _L)] = b1 + 1
    idx_v[pl.ds(2 * _L, _L)] = b1 + 2
    idx_v[pl.ds(3 * _L, _L)] = b2
    idx_v[pl.ds(4 * _L, _L)] = b2 + 1
    idx_v[pl.ds(5 * _L, _L)] = b2 + 2

    pltpu.async_copy(coords_hbm.at[idx_v], gat_v, sem).wait()

    dx = gat_v[pl.ds(0, _L)] - gat_v[pl.ds(3 * _L, _L)]
    dy = gat_v[pl.ds(_L, _L)] - gat_v[pl.ds(4 * _L, _L)]
    dz = gat_v[pl.ds(2 * _L, _L)] - gat_v[pl.ds(5 * _L, _L)]
    d2 = dx * dx + dy * dy + dz * dz

    bits = plsc.bitcast(d2, jnp.int32)
    y = plsc.bitcast(jnp.int32(0x5F3759DF) - (bits >> 1), jnp.float32)
    half_d2 = 0.5 * d2
    for _unused in range(3):
        y = y * (1.5 - half_d2 * y * y)
    dist = d2 * y

    h = dist - _IDEAL
    h = h * h
    h = jnp.where(iota < _N_CNSTRS, h, 0.0)
    total = jnp.sum(h)

    out_v[...] = lax.broadcast(total, (_L,))
    pltpu.sync_copy(out_v, out_hbm)


def kernel(coords, pose_stack_block_coord_offset):
    n_atoms = coords.shape[1]
    n_blocks = pose_stack_block_coord_offset.shape[1]
    c = coords.reshape(n_atoms * 3)
    off = pose_stack_block_coord_offset.reshape(n_blocks)

    mesh = plsc.VectorSubcoreMesh(
        core_axis_name="c", subcore_axis_name="s", num_cores=1, num_subcores=1
    )
    res = pl.kernel(
        _sc_body,
        out_type=jax.ShapeDtypeStruct((_L,), jnp.float32),
        mesh=mesh,
        compiler_params=pltpu.CompilerParams(needs_layout_passes=False),
        scratch_types=[
            pltpu.VMEM((32,), jnp.int32),
            pltpu.VMEM((6 * _L,), jnp.int32),
            pltpu.VMEM((6 * _L,), jnp.float32),
            pltpu.VMEM((_L,), jnp.float32),
            pltpu.SemaphoreType.DMA,
        ],
    )(c, off)
    return res[:1].reshape(1, 1)
